# Initial kernel scaffold; baseline (speedup 1.0000x reference)
import jax
import jax.numpy as jnp
from jax.experimental import pallas as pl


def kernel(A, X, gru_Wih, gru_Whh, gru_bih, gru_bhh, W1, b1, W2, b2, Wlin, blin):
    raise NotImplementedError("write your pallas kernel here")


# speedup vs baseline: 2.9346x; 2.9346x over previous
import functools

import jax
import jax.numpy as jnp
from jax.experimental import pallas as pl


def _leaky(x):
    return jnp.where(x >= 0, x, 0.01 * x)


def _gru_body(T, F, H, x_ref, wih_ref, whh_ref, bih_ref, bhh_ref, o_ref):
    x = x_ref[0]
    wih = wih_ref[...]
    whh = whh_ref[...]
    bih = bih_ref[0]
    bhh = bhh_ref[0]
    h = None
    for t in range(T):
        xt = x[:, t * F:(t + 1) * F]
        gi = jnp.dot(xt, wih, preferred_element_type=jnp.float32) + bih
        if h is None:
            gh = jnp.broadcast_to(bhh, gi.shape)
        else:
            gh = jnp.dot(h, whh, preferred_element_type=jnp.float32) + bhh
        r = jax.nn.sigmoid(gi[:, :H] + gh[:, :H])
        z = jax.nn.sigmoid(gi[:, H:2 * H] + gh[:, H:2 * H])
        n = jnp.tanh(gi[:, 2 * H:] + r * gh[:, 2 * H:])
        if h is None:
            h = (1.0 - z) * n
        else:
            h = (1.0 - z) * n + z * h
    o_ref[...] = h


def _gcn_body(a_ref, y_ref, w_ref, b_ref, o_ref):
    s = jnp.dot(a_ref[...], y_ref[...], preferred_element_type=jnp.float32)
    t = jnp.dot(s, w_ref[...], preferred_element_type=jnp.float32) + b_ref[0]
    o_ref[...] = _leaky(t)


def _gcn_head_body(a_ref, y_ref, w_ref, b_ref, wlin_ref, blin_ref, o_ref):
    s = jnp.dot(a_ref[...], y_ref[...], preferred_element_type=jnp.float32)
    t = jnp.dot(s, w_ref[...], preferred_element_type=jnp.float32) + b_ref[0]
    t = _leaky(t)
    o_ref[0] = jnp.dot(t, wlin_ref[...], preferred_element_type=jnp.float32) + blin_ref[0]


def kernel(A, X, gru_Wih, gru_Whh, gru_bih, gru_bhh, W1, b1, W2, b2, Wlin, blin):
    B, N, T, F = X.shape
    H = gru_Whh.shape[1]
    T_OUT = Wlin.shape[1]

    Xr = X.reshape(B, N, T * F)
    wih_t = gru_Wih.T
    whh_t = gru_Whh.T
    bih2 = gru_bih.reshape(1, -1)
    bhh2 = gru_bhh.reshape(1, -1)

    BM_G = min(N, 1024)
    BM_A = min(N, 256)

    h_nb = pl.pallas_call(
        functools.partial(_gru_body, T, F, H),
        grid=(B, N // BM_G),
        in_specs=[
            pl.BlockSpec((1, BM_G, T * F), lambda b, j: (b, j, 0)),
            pl.BlockSpec((F, 3 * H), lambda b, j: (0, 0)),
            pl.BlockSpec((H, 3 * H), lambda b, j: (0, 0)),
            pl.BlockSpec((1, 3 * H), lambda b, j: (0, 0)),
            pl.BlockSpec((1, 3 * H), lambda b, j: (0, 0)),
        ],
        out_specs=pl.BlockSpec((BM_G, H), lambda b, j: (j, b)),
        out_shape=jax.ShapeDtypeStruct((N, B * H), jnp.float32),
    )(Xr, wih_t, whh_t, bih2, bhh2)

    t2 = pl.pallas_call(
        _gcn_body,
        grid=(B, N // BM_A),
        in_specs=[
            pl.BlockSpec((BM_A, N), lambda b, j: (j, 0)),
            pl.BlockSpec((N, H), lambda b, j: (0, b)),
            pl.BlockSpec((H, H), lambda b, j: (0, 0)),
            pl.BlockSpec((1, H), lambda b, j: (0, 0)),
        ],
        out_specs=pl.BlockSpec((BM_A, H), lambda b, j: (j, b)),
        out_shape=jax.ShapeDtypeStruct((N, B * H), jnp.float32),
    )(A, h_nb, W1, b1.reshape(1, -1))

    out = pl.pallas_call(
        _gcn_head_body,
        grid=(B, N // BM_A),
        in_specs=[
            pl.BlockSpec((BM_A, N), lambda b, j: (j, 0)),
            pl.BlockSpec((N, H), lambda b, j: (0, b)),
            pl.BlockSpec((H, H), lambda b, j: (0, 0)),
            pl.BlockSpec((1, H), lambda b, j: (0, 0)),
            pl.BlockSpec((H, T_OUT), lambda b, j: (0, 0)),
            pl.BlockSpec((1, T_OUT), lambda b, j: (0, 0)),
        ],
        out_specs=pl.BlockSpec((1, BM_A, T_OUT), lambda b, j: (b, j, 0)),
        out_shape=jax.ShapeDtypeStruct((B, N, T_OUT), jnp.float32),
    )(A, t2, W2, b2.reshape(1, -1), Wlin, blin.reshape(1, -1))

    return out


# speedup vs baseline: 3.2363x; 1.1028x over previous
import functools

import jax
import jax.numpy as jnp
from jax.experimental import pallas as pl


def _leaky(x):
    return jnp.where(x >= 0, x, 0.01 * x)


def _gru_body(T, F, H, x_ref, wih_ref, whh_ref, bih_ref, bhh_ref, o_ref):
    x = x_ref[0]
    wih = wih_ref[...]
    whh = whh_ref[...]
    bih = bih_ref[0]
    bhh = bhh_ref[0]
    h = None
    for t in range(T):
        xt = x[:, t * F:(t + 1) * F]
        gi = jnp.dot(xt, wih, preferred_element_type=jnp.float32) + bih
        if h is None:
            gh = jnp.broadcast_to(bhh, gi.shape)
        else:
            gh = jnp.dot(h.astype(jnp.bfloat16), whh,
                         preferred_element_type=jnp.float32) + bhh
        r = jax.nn.sigmoid(gi[:, :H] + gh[:, :H])
        z = jax.nn.sigmoid(gi[:, H:2 * H] + gh[:, H:2 * H])
        n = jnp.tanh(gi[:, 2 * H:] + r * gh[:, 2 * H:])
        if h is None:
            h = (1.0 - z) * n
        else:
            h = (1.0 - z) * n + z * h
    o_ref[...] = h.astype(jnp.bfloat16)


def _gcn_body(a_ref, y_ref, w_ref, b_ref, o_ref):
    s = jnp.dot(a_ref[...], y_ref[...], preferred_element_type=jnp.float32)
    t = jnp.dot(s.astype(jnp.bfloat16), w_ref[...],
                preferred_element_type=jnp.float32) + b_ref[0]
    o_ref[...] = _leaky(t).astype(jnp.bfloat16)


def _gcn_head_body(a_ref, y_ref, w_ref, b_ref, wlin_ref, blin_ref, o_ref):
    s = jnp.dot(a_ref[...], y_ref[...], preferred_element_type=jnp.float32)
    t = jnp.dot(s.astype(jnp.bfloat16), w_ref[...],
                preferred_element_type=jnp.float32) + b_ref[0]
    t = _leaky(t)
    o_ref[0] = jnp.dot(t.astype(jnp.bfloat16), wlin_ref[...],
                       preferred_element_type=jnp.float32) + blin_ref[0]


def kernel(A, X, gru_Wih, gru_Whh, gru_bih, gru_bhh, W1, b1, W2, b2, Wlin, blin):
    B, N, T, F = X.shape
    H = gru_Whh.shape[1]
    T_OUT = Wlin.shape[1]

    Xr = X.reshape(B, N, T * F).astype(jnp.bfloat16)
    Abf = A.astype(jnp.bfloat16)
    wih_t = gru_Wih.T.astype(jnp.bfloat16)
    whh_t = gru_Whh.T.astype(jnp.bfloat16)
    bih2 = gru_bih.reshape(1, -1)
    bhh2 = gru_bhh.reshape(1, -1)

    BM_G = min(N, 1024)
    BM_A = min(N, 256)

    h_nb = pl.pallas_call(
        functools.partial(_gru_body, T, F, H),
        grid=(B, N // BM_G),
        in_specs=[
            pl.BlockSpec((1, BM_G, T * F), lambda b, j: (b, j, 0)),
            pl.BlockSpec((F, 3 * H), lambda b, j: (0, 0)),
            pl.BlockSpec((H, 3 * H), lambda b, j: (0, 0)),
            pl.BlockSpec((1, 3 * H), lambda b, j: (0, 0)),
            pl.BlockSpec((1, 3 * H), lambda b, j: (0, 0)),
        ],
        out_specs=pl.BlockSpec((BM_G, H), lambda b, j: (j, b)),
        out_shape=jax.ShapeDtypeStruct((N, B * H), jnp.bfloat16),
    )(Xr, wih_t, whh_t, bih2, bhh2)

    t2 = pl.pallas_call(
        _gcn_body,
        grid=(B, N // BM_A),
        in_specs=[
            pl.BlockSpec((BM_A, N), lambda b, j: (j, 0)),
            pl.BlockSpec((N, H), lambda b, j: (0, b)),
            pl.BlockSpec((H, H), lambda b, j: (0, 0)),
            pl.BlockSpec((1, H), lambda b, j: (0, 0)),
        ],
        out_specs=pl.BlockSpec((BM_A, H), lambda b, j: (j, b)),
        out_shape=jax.ShapeDtypeStruct((N, B * H), jnp.bfloat16),
    )(Abf, h_nb, W1.astype(jnp.bfloat16), b1.reshape(1, -1))

    out = pl.pallas_call(
        _gcn_head_body,
        grid=(B, N // BM_A),
        in_specs=[
            pl.BlockSpec((BM_A, N), lambda b, j: (j, 0)),
            pl.BlockSpec((N, H), lambda b, j: (0, b)),
            pl.BlockSpec((H, H), lambda b, j: (0, 0)),
            pl.BlockSpec((1, H), lambda b, j: (0, 0)),
            pl.BlockSpec((H, T_OUT), lambda b, j: (0, 0)),
            pl.BlockSpec((1, T_OUT), lambda b, j: (0, 0)),
        ],
        out_specs=pl.BlockSpec((1, BM_A, T_OUT), lambda b, j: (b, j, 0)),
        out_shape=jax.ShapeDtypeStruct((B, N, T_OUT), jnp.float32),
    )(Abf, t2, W2.astype(jnp.bfloat16), b2.reshape(1, -1),
      Wlin.astype(jnp.bfloat16), blin.reshape(1, -1))

    return out


# speedup vs baseline: 3.2697x; 1.0103x over previous
import functools

import jax
import jax.numpy as jnp
from jax.experimental import pallas as pl


def _leaky(x):
    return jnp.where(x >= 0, x, 0.01 * x)


def _gru_body(T, F, H, x_ref, wih_ref, whh_ref, bih_ref, bhh_ref, o_ref):
    x = x_ref[0]
    wih = wih_ref[...]
    whh = whh_ref[...]
    bih = bih_ref[0]
    bhh = bhh_ref[0]
    h = None
    for t in range(T):
        xt = x[:, t * F:(t + 1) * F]
        gi = jnp.dot(xt, wih, preferred_element_type=jnp.float32) + bih
        if h is None:
            gh = jnp.broadcast_to(bhh, gi.shape)
        else:
            gh = jnp.dot(h.astype(jnp.bfloat16), whh,
                         preferred_element_type=jnp.float32) + bhh
        r = 0.5 * jnp.tanh(0.5 * (gi[:, :H] + gh[:, :H])) + 0.5
        z = 0.5 * jnp.tanh(0.5 * (gi[:, H:2 * H] + gh[:, H:2 * H])) + 0.5
        n = jnp.tanh(gi[:, 2 * H:] + r * gh[:, 2 * H:])
        if h is None:
            h = (1.0 - z) * n
        else:
            h = (1.0 - z) * n + z * h
    o_ref[...] = h.astype(jnp.bfloat16)


def _gcn_body(a_ref, y_ref, w_ref, b_ref, o_ref):
    s = jnp.dot(a_ref[...], y_ref[...], preferred_element_type=jnp.float32)
    t = jnp.dot(s.astype(jnp.bfloat16), w_ref[...],
                preferred_element_type=jnp.float32) + b_ref[0]
    o_ref[...] = _leaky(t).astype(jnp.bfloat16)


def _gcn_head_body(a_ref, y_ref, w_ref, b_ref, wlin_ref, blin_ref, o_ref):
    s = jnp.dot(a_ref[...], y_ref[...], preferred_element_type=jnp.float32)
    t = jnp.dot(s.astype(jnp.bfloat16), w_ref[...],
                preferred_element_type=jnp.float32) + b_ref[0]
    t = _leaky(t)
    o_ref[0] = jnp.dot(t.astype(jnp.bfloat16), wlin_ref[...],
                       preferred_element_type=jnp.float32) + blin_ref[0]


def kernel(A, X, gru_Wih, gru_Whh, gru_bih, gru_bhh, W1, b1, W2, b2, Wlin, blin):
    B, N, T, F = X.shape
    H = gru_Whh.shape[1]
    T_OUT = Wlin.shape[1]

    Xr = X.reshape(B, N, T * F).astype(jnp.bfloat16)
    Abf = A.astype(jnp.bfloat16)
    wih_t = gru_Wih.T.astype(jnp.bfloat16)
    whh_t = gru_Whh.T.astype(jnp.bfloat16)
    bih2 = gru_bih.reshape(1, -1)
    bhh2 = gru_bhh.reshape(1, -1)

    BM_G = min(N, 1024)
    BM_A = min(N, 256)

    h_nb = pl.pallas_call(
        functools.partial(_gru_body, T, F, H),
        grid=(B, N // BM_G),
        in_specs=[
            pl.BlockSpec((1, BM_G, T * F), lambda b, j: (b, j, 0)),
            pl.BlockSpec((F, 3 * H), lambda b, j: (0, 0)),
            pl.BlockSpec((H, 3 * H), lambda b, j: (0, 0)),
            pl.BlockSpec((1, 3 * H), lambda b, j: (0, 0)),
            pl.BlockSpec((1, 3 * H), lambda b, j: (0, 0)),
        ],
        out_specs=pl.BlockSpec((BM_G, H), lambda b, j: (j, b)),
        out_shape=jax.ShapeDtypeStruct((N, B * H), jnp.bfloat16),
    )(Xr, wih_t, whh_t, bih2, bhh2)

    t2 = pl.pallas_call(
        _gcn_body,
        grid=(B, N // BM_A),
        in_specs=[
            pl.BlockSpec((BM_A, N), lambda b, j: (j, 0)),
            pl.BlockSpec((N, H), lambda b, j: (0, b)),
            pl.BlockSpec((H, H), lambda b, j: (0, 0)),
            pl.BlockSpec((1, H), lambda b, j: (0, 0)),
        ],
        out_specs=pl.BlockSpec((BM_A, H), lambda b, j: (j, b)),
        out_shape=jax.ShapeDtypeStruct((N, B * H), jnp.bfloat16),
    )(Abf, h_nb, W1.astype(jnp.bfloat16), b1.reshape(1, -1))

    out = pl.pallas_call(
        _gcn_head_body,
        grid=(B, N // BM_A),
        in_specs=[
            pl.BlockSpec((BM_A, N), lambda b, j: (j, 0)),
            pl.BlockSpec((N, H), lambda b, j: (0, b)),
            pl.BlockSpec((H, H), lambda b, j: (0, 0)),
            pl.BlockSpec((1, H), lambda b, j: (0, 0)),
            pl.BlockSpec((H, T_OUT), lambda b, j: (0, 0)),
            pl.BlockSpec((1, T_OUT), lambda b, j: (0, 0)),
        ],
        out_specs=pl.BlockSpec((1, BM_A, T_OUT), lambda b, j: (b, j, 0)),
        out_shape=jax.ShapeDtypeStruct((B, N, T_OUT), jnp.float32),
    )(Abf, t2, W2.astype(jnp.bfloat16), b2.reshape(1, -1),
      Wlin.astype(jnp.bfloat16), blin.reshape(1, -1))

    return out


# speedup vs baseline: 3.2718x; 1.0006x over previous
import functools

import jax
import jax.numpy as jnp
from jax.experimental import pallas as pl
from jax.experimental.pallas import tpu as pltpu


def _leaky(x):
    return jnp.where(x >= 0, x, 0.01 * x)


def _gru_body(T, F, H, x_ref, wih_ref, whh_ref, bih_ref, bhh_ref, o_ref):
    x = x_ref[0]
    wih = wih_ref[...]
    whh = whh_ref[...]
    bih = bih_ref[0]
    bhh = bhh_ref[0]
    h = None
    for t in range(T):
        xt = x[:, t * F:(t + 1) * F]
        gi = jnp.dot(xt, wih, preferred_element_type=jnp.float32) + bih
        if h is None:
            gh = jnp.broadcast_to(bhh, gi.shape)
        else:
            gh = jnp.dot(h.astype(jnp.bfloat16), whh,
                         preferred_element_type=jnp.float32) + bhh
        r = 0.5 * jnp.tanh(0.5 * (gi[:, :H] + gh[:, :H])) + 0.5
        z = 0.5 * jnp.tanh(0.5 * (gi[:, H:2 * H] + gh[:, H:2 * H])) + 0.5
        n = jnp.tanh(gi[:, 2 * H:] + r * gh[:, 2 * H:])
        if h is None:
            h = (1.0 - z) * n
        else:
            h = (1.0 - z) * n + z * h
    o_ref[...] = h.astype(jnp.bfloat16)


def _gcn_body(a_ref, y_ref, w_ref, b_ref, o_ref):
    s = jnp.dot(a_ref[...], y_ref[...], preferred_element_type=jnp.float32)
    t = jnp.dot(s.astype(jnp.bfloat16), w_ref[...],
                preferred_element_type=jnp.float32) + b_ref[0]
    o_ref[...] = _leaky(t).astype(jnp.bfloat16)


def _gcn_head_body(a_ref, y_ref, w_ref, b_ref, wlin_ref, blin_ref, o_ref):
    s = jnp.dot(a_ref[...], y_ref[...], preferred_element_type=jnp.float32)
    t = jnp.dot(s.astype(jnp.bfloat16), w_ref[...],
                preferred_element_type=jnp.float32) + b_ref[0]
    t = _leaky(t)
    o_ref[0] = jnp.dot(t.astype(jnp.bfloat16), wlin_ref[...],
                       preferred_element_type=jnp.float32) + blin_ref[0]


def kernel(A, X, gru_Wih, gru_Whh, gru_bih, gru_bhh, W1, b1, W2, b2, Wlin, blin):
    B, N, T, F = X.shape
    H = gru_Whh.shape[1]
    T_OUT = Wlin.shape[1]

    Xr = X.reshape(B, N, T * F).astype(jnp.bfloat16)
    Abf = A.astype(jnp.bfloat16)
    wih_t = gru_Wih.T.astype(jnp.bfloat16)
    whh_t = gru_Whh.T.astype(jnp.bfloat16)
    bih2 = gru_bih.reshape(1, -1)
    bhh2 = gru_bhh.reshape(1, -1)

    BM_G = min(N, 1024)
    BM_A = min(N, 256)

    h_nb = pl.pallas_call(
        functools.partial(_gru_body, T, F, H),
        grid=(B, N // BM_G),
        in_specs=[
            pl.BlockSpec((1, BM_G, T * F), lambda b, j: (b, j, 0)),
            pl.BlockSpec((F, 3 * H), lambda b, j: (0, 0)),
            pl.BlockSpec((H, 3 * H), lambda b, j: (0, 0)),
            pl.BlockSpec((1, 3 * H), lambda b, j: (0, 0)),
            pl.BlockSpec((1, 3 * H), lambda b, j: (0, 0)),
        ],
        out_specs=pl.BlockSpec((BM_G, H), lambda b, j: (j, b)),
        out_shape=jax.ShapeDtypeStruct((N, B * H), jnp.bfloat16),
        compiler_params=pltpu.CompilerParams(
            dimension_semantics=("parallel", "parallel")),
    )(Xr, wih_t, whh_t, bih2, bhh2)

    t2 = pl.pallas_call(
        _gcn_body,
        grid=(B, N // BM_A),
        in_specs=[
            pl.BlockSpec((BM_A, N), lambda b, j: (j, 0)),
            pl.BlockSpec((N, H), lambda b, j: (0, b)),
            pl.BlockSpec((H, H), lambda b, j: (0, 0)),
            pl.BlockSpec((1, H), lambda b, j: (0, 0)),
        ],
        out_specs=pl.BlockSpec((BM_A, H), lambda b, j: (j, b)),
        out_shape=jax.ShapeDtypeStruct((N, B * H), jnp.bfloat16),
        compiler_params=pltpu.CompilerParams(
            dimension_semantics=("parallel", "parallel")),
    )(Abf, h_nb, W1.astype(jnp.bfloat16), b1.reshape(1, -1))

    out = pl.pallas_call(
        _gcn_head_body,
        grid=(B, N // BM_A),
        in_specs=[
            pl.BlockSpec((BM_A, N), lambda b, j: (j, 0)),
            pl.BlockSpec((N, H), lambda b, j: (0, b)),
            pl.BlockSpec((H, H), lambda b, j: (0, 0)),
            pl.BlockSpec((1, H), lambda b, j: (0, 0)),
            pl.BlockSpec((H, T_OUT), lambda b, j: (0, 0)),
            pl.BlockSpec((1, T_OUT), lambda b, j: (0, 0)),
        ],
        out_specs=pl.BlockSpec((1, BM_A, T_OUT), lambda b, j: (b, j, 0)),
        out_shape=jax.ShapeDtypeStruct((B, N, T_OUT), jnp.float32),
        compiler_params=pltpu.CompilerParams(
            dimension_semantics=("parallel", "parallel")),
    )(Abf, t2, W2.astype(jnp.bfloat16), b2.reshape(1, -1),
      Wlin.astype(jnp.bfloat16), blin.reshape(1, -1))

    return out


# speedup vs baseline: 4.8544x; 1.4837x over previous
import functools

import jax
import jax.numpy as jnp
from jax.experimental import pallas as pl
from jax.experimental.pallas import tpu as pltpu


def _leaky(x):
    return jnp.where(x >= 0, x, 0.01 * x)


def _gru_body(T, F, H, B, x_ref, wih_ref, whh_ref, bih_ref, bhh_ref, w1_ref,
              o_ref):
    x = x_ref[0]
    wih = wih_ref[...]
    whh = whh_ref[...]
    bih = bih_ref[0]
    bhh = bhh_ref[0]
    h = None
    for t in range(T):
        xt = x[:, t * F:(t + 1) * F]
        gi = jnp.dot(xt, wih, preferred_element_type=jnp.float32) + bih
        if h is None:
            gh = jnp.broadcast_to(bhh, gi.shape)
        else:
            gh = jnp.dot(h.astype(jnp.bfloat16), whh,
                         preferred_element_type=jnp.float32) + bhh
        r = 0.5 * jnp.tanh(0.5 * (gi[:, :H] + gh[:, :H])) + 0.5
        z = 0.5 * jnp.tanh(0.5 * (gi[:, H:2 * H] + gh[:, H:2 * H])) + 0.5
        n = jnp.tanh(gi[:, 2 * H:] + r * gh[:, 2 * H:])
        if h is None:
            h = (1.0 - z) * n
        else:
            h = (1.0 - z) * n + z * h
    y1 = jnp.dot(h.astype(jnp.bfloat16), w1_ref[...],
                 preferred_element_type=jnp.float32)
    o_ref[...] = y1.astype(jnp.bfloat16)


def _gcn1_body(B, H, a_ref, y_ref, b_ref, w2_ref, o_ref):
    u = jnp.dot(a_ref[...], y_ref[...], preferred_element_type=jnp.float32)
    t2 = _leaky(u + b_ref[0])
    w2 = w2_ref[...]
    for b in range(B):
        yb = jnp.dot(t2[:, b * H:(b + 1) * H].astype(jnp.bfloat16), w2,
                     preferred_element_type=jnp.float32)
        o_ref[:, b * H:(b + 1) * H] = yb.astype(jnp.bfloat16)


def _gcn2_body(B, H, a_ref, y_ref, b_ref, wlin_ref, blin_ref, o_ref):
    v = jnp.dot(a_ref[...], y_ref[...], preferred_element_type=jnp.float32)
    t3 = _leaky(v + b_ref[0])
    wlin = wlin_ref[...]
    blin = blin_ref[0]
    for b in range(B):
        ob = jnp.dot(t3[:, b * H:(b + 1) * H].astype(jnp.bfloat16), wlin,
                     preferred_element_type=jnp.float32) + blin
        o_ref[b] = ob


def kernel(A, X, gru_Wih, gru_Whh, gru_bih, gru_bhh, W1, b1, W2, b2, Wlin, blin):
    B, N, T, F = X.shape
    H = gru_Whh.shape[1]
    T_OUT = Wlin.shape[1]

    Xr = X.reshape(B, N, T * F).astype(jnp.bfloat16)
    Abf = A.astype(jnp.bfloat16)
    wih_t = gru_Wih.T.astype(jnp.bfloat16)
    whh_t = gru_Whh.T.astype(jnp.bfloat16)
    bih2 = gru_bih.reshape(1, -1)
    bhh2 = gru_bhh.reshape(1, -1)
    b1t = jnp.tile(b1, B).reshape(1, B * H)
    b2t = jnp.tile(b2, B).reshape(1, B * H)

    BM_G = min(N, 1024)
    BM_A = min(N, 256)

    y1 = pl.pallas_call(
        functools.partial(_gru_body, T, F, H, B),
        grid=(B, N // BM_G),
        in_specs=[
            pl.BlockSpec((1, BM_G, T * F), lambda b, j: (b, j, 0)),
            pl.BlockSpec((F, 3 * H), lambda b, j: (0, 0)),
            pl.BlockSpec((H, 3 * H), lambda b, j: (0, 0)),
            pl.BlockSpec((1, 3 * H), lambda b, j: (0, 0)),
            pl.BlockSpec((1, 3 * H), lambda b, j: (0, 0)),
            pl.BlockSpec((H, H), lambda b, j: (0, 0)),
        ],
        out_specs=pl.BlockSpec((BM_G, H), lambda b, j: (j, b)),
        out_shape=jax.ShapeDtypeStruct((N, B * H), jnp.bfloat16),
        compiler_params=pltpu.CompilerParams(
            dimension_semantics=("parallel", "parallel")),
    )(Xr, wih_t, whh_t, bih2, bhh2, W1.astype(jnp.bfloat16))

    y2 = pl.pallas_call(
        functools.partial(_gcn1_body, B, H),
        grid=(N // BM_A,),
        in_specs=[
            pl.BlockSpec((BM_A, N), lambda j: (j, 0)),
            pl.BlockSpec((N, B * H), lambda j: (0, 0)),
            pl.BlockSpec((1, B * H), lambda j: (0, 0)),
            pl.BlockSpec((H, H), lambda j: (0, 0)),
        ],
        out_specs=pl.BlockSpec((BM_A, B * H), lambda j: (j, 0)),
        out_shape=jax.ShapeDtypeStruct((N, B * H), jnp.bfloat16),
        compiler_params=pltpu.CompilerParams(
            dimension_semantics=("parallel",)),
    )(Abf, y1, b1t, W2.astype(jnp.bfloat16))

    out = pl.pallas_call(
        functools.partial(_gcn2_body, B, H),
        grid=(N // BM_A,),
        in_specs=[
            pl.BlockSpec((BM_A, N), lambda j: (j, 0)),
            pl.BlockSpec((N, B * H), lambda j: (0, 0)),
            pl.BlockSpec((1, B * H), lambda j: (0, 0)),
            pl.BlockSpec((H, T_OUT), lambda j: (0, 0)),
            pl.BlockSpec((1, T_OUT), lambda j: (0, 0)),
        ],
        out_specs=pl.BlockSpec((B, BM_A, T_OUT), lambda j: (0, j, 0)),
        out_shape=jax.ShapeDtypeStruct((B, N, T_OUT), jnp.float32),
        compiler_params=pltpu.CompilerParams(
            dimension_semantics=("parallel",)),
    )(Abf, y2, b2t, Wlin.astype(jnp.bfloat16), blin.reshape(1, -1))

    return out
